# baseline (speedup 1.0000x reference)
import functools

import jax
import jax.numpy as jnp
from jax import lax
from jax.experimental import pallas as pl
from jax.experimental.pallas import tpu as pltpu
from jax.experimental.pallas import tpu_sc as plsc

_NC = 2
_NS = 16
_NW = _NC * _NS
_C = 112
_NB = 3
_NI = 6


def _sc_mesh():
    return plsc.VectorSubcoreMesh(core_axis_name="c", subcore_axis_name="s")


def _make_deg_kernel(Ep, Np):
    ept = Ep // _NW
    nv = ept // 16

    @functools.partial(
        pl.kernel,
        mesh=_sc_mesh(),
        compiler_params=pltpu.CompilerParams(needs_layout_passes=False),
        out_type=jax.ShapeDtypeStruct((_NW * Np,), jnp.float32),
        scratch_types=[
            pltpu.VMEM((ept,), jnp.int32),
            pltpu.VMEM((Np,), jnp.float32),
            pltpu.SemaphoreType.DMA,
        ],
    )
    def deg_kernel(dst_hbm, out_hbm, didx, hist, sem):
        cid = lax.axis_index("c")
        sid = lax.axis_index("s")
        wid = cid * _NS + sid
        pltpu.async_copy(dst_hbm.at[pl.ds(wid * ept, ept)], didx, sem).wait()
        zeros = jnp.zeros((16,), jnp.float32)

        def zbody(i, _):
            hist[pl.ds(i * 16, 16)] = zeros
            return 0

        lax.fori_loop(0, Np // 16, zbody, 0)

        def body(j, _):
            v = didx[pl.ds(j * 16, 16)]
            cnt, m = plsc.scan_count(v)
            plsc.addupdate_scatter(hist, [v], cnt.astype(jnp.float32), mask=m)
            return 0

        lax.fori_loop(0, nv, body, 0)
        pltpu.sync_copy(hist, out_hbm.at[pl.ds(wid * Np, Np)])

    return deg_kernel


def _make_agg_kernel(Ep, Np, H):
    ept = Ep // _NW
    nch = ept // _C
    rpt = Np // _NS

    @functools.partial(
        pl.kernel,
        mesh=_sc_mesh(),
        out_type=jax.ShapeDtypeStruct((_NC, Np, H), jnp.float32),
        scratch_types=[
            pltpu.VMEM((_NI, _C), jnp.int32),
            pltpu.VMEM((_NI, _C), jnp.int32),
            pltpu.VMEM((_NB, _C, H), jnp.float32),
            pltpu.VMEM_SHARED((Np, H), jnp.float32),
            [pltpu.SemaphoreType.DMA] * _NI,
            [pltpu.SemaphoreType.DMA] * _NI,
            [pltpu.SemaphoreType.DMA] * _NB,
        ],
    )
    def agg_kernel(u_hbm, src_hbm, dst_hbm, zeros_hbm, out_hbm,
                   sidx, didx, rows, acc, isems, dsems, gsems):
        cid = lax.axis_index("c")
        sid = lax.axis_index("s")
        wid = cid * _NS + sid
        row0 = sid * rpt

        ebase = wid * ept

        def load_idx(ci, slot):
            pltpu.async_copy(src_hbm.at[pl.ds(ebase + ci * _C, _C)],
                             sidx.at[slot], isems[slot])
            pltpu.async_copy(dst_hbm.at[pl.ds(ebase + ci * _C, _C)],
                             didx.at[slot], dsems[slot])

        def wait_sidx(ci, slot):
            pltpu.make_async_copy(src_hbm.at[pl.ds(ebase + ci * _C, _C)],
                                  sidx.at[slot], isems[slot]).wait()

        def wait_didx(ci, slot):
            pltpu.make_async_copy(dst_hbm.at[pl.ds(ebase + ci * _C, _C)],
                                  didx.at[slot], dsems[slot]).wait()

        def fire_gather(slot, gb):
            pltpu.async_copy(u_hbm.at[sidx.at[slot]], rows.at[gb], gsems[gb])

        def wait_gather(slot, gb):
            pltpu.make_async_copy(u_hbm.at[sidx.at[slot]], rows.at[gb],
                                  gsems[gb]).wait()

        for k in range(_NI - 1):
            load_idx(k, k)
        pltpu.sync_copy(zeros_hbm, acc.at[pl.ds(row0, rpt)])
        plsc.subcore_barrier()
        for k in range(_NB - 1):
            wait_sidx(k, k)
            fire_gather(k, k % _NB)

        def grp(g, _):
            for k in range(_NI):
                ci = g * _NI + k

                @pl.when(ci + _NI - 1 < nch)
                def _():
                    load_idx(ci + _NI - 1, (k + _NI - 1) % _NI)

                @pl.when(ci + _NB - 1 < nch)
                def _():
                    wait_sidx(ci + _NB - 1, (k + _NB - 1) % _NI)
                    fire_gather((k + _NB - 1) % _NI, (k + _NB - 1) % _NB)
                wait_gather(k % _NI, k % _NB)
                wait_didx(ci, k % _NI)
                pltpu.sync_copy(rows.at[k % _NB], acc.at[didx.at[k % _NI]],
                                add=True)
            return 0

        lax.fori_loop(0, nch // _NI, grp, 0)
        plsc.subcore_barrier()
        pltpu.sync_copy(acc.at[pl.ds(row0, rpt)], out_hbm.at[cid, pl.ds(row0, rpt)])

    return agg_kernel


def _tc1_body(x_ref, w_ref, hists_ref, ones_ref, u_ref, s_ref, dinv_ref):
    deg = lax.dot_general(hists_ref[...], ones_ref[...],
                          (((0,), (0,)), ((), ())),
                          preferred_element_type=jnp.float32) + 1.0
    dinv = lax.rsqrt(deg)
    h = jnp.dot(x_ref[...], w_ref[...], preferred_element_type=jnp.float32)
    u = h * dinv
    u_ref[...] = u
    s_ref[...] = u * dinv
    dinv_ref[...] = dinv


def _tc2_body(aggp_ref, s_ref, dinv_ref, b_ref, w_ref, u_ref, s2_ref):
    dinv = dinv_ref[...]
    z = dinv * (aggp_ref[0] + aggp_ref[1]) + s_ref[...] + b_ref[...]
    r = jnp.maximum(z, 0.0)
    h = jnp.dot(r, w_ref[...], preferred_element_type=jnp.float32)
    u = h * dinv
    u_ref[...] = u
    s2_ref[...] = u * dinv


def _tc3_body(n_real, aggp_ref, s_ref, dinv_ref, b_ref, wfc_ref, bfc_ref, out_ref):
    z = dinv_ref[...] * (aggp_ref[0] + aggp_ref[1]) + s_ref[...] + b_ref[...]
    r = jnp.maximum(z, 0.0)
    rows = lax.broadcasted_iota(jnp.int32, r.shape, 0)
    r = jnp.where(rows < n_real, r, 0.0)
    g = jnp.sum(r, axis=0, keepdims=True) * (1.0 / n_real)
    out_ref[...] = jnp.dot(g, wfc_ref[...],
                           preferred_element_type=jnp.float32) + bfc_ref[...]


def kernel(x, edge_index, W1, b1, W2, b2, Wfc, bfc):
    N, D = x.shape
    H = W1.shape[1]
    O = Wfc.shape[1]
    E = edge_index.shape[1]

    Np = ((N + 1 + 127) // 128) * 128
    blk = _NW * _C * _NB
    Ep = ((E + blk - 1) // blk) * blk
    rpt = Np // _NS
    nch = Ep // (_NW * _C)

    npad = Ep - E
    pad_src = jnp.full((npad,), N, dtype=jnp.int32)
    pad_dst = N + (jnp.arange(npad, dtype=jnp.int32) % (Np - N))
    src = jnp.concatenate([edge_index[0], pad_src])
    dst = jnp.concatenate([edge_index[1], pad_dst])
    x_pad = jnp.zeros((Np, D), dtype=jnp.float32).at[:N].set(x)
    ones32 = jnp.ones((_NW, 1), dtype=jnp.float32)
    zerosH = jnp.zeros((rpt, H), dtype=jnp.float32)

    hists = _make_deg_kernel(Ep, Np)(dst).reshape(_NW, Np)

    tc1 = pl.pallas_call(
        _tc1_body,
        out_shape=(jax.ShapeDtypeStruct((Np, H), jnp.float32),
                   jax.ShapeDtypeStruct((Np, H), jnp.float32),
                   jax.ShapeDtypeStruct((Np, 1), jnp.float32)),
    )
    u1, s1, dinv = tc1(x_pad, W1, hists, ones32)

    agg = _make_agg_kernel(Ep, Np, H)
    agg1 = agg(u1, src, dst, zerosH)

    tc2 = pl.pallas_call(
        _tc2_body,
        out_shape=(jax.ShapeDtypeStruct((Np, H), jnp.float32),
                   jax.ShapeDtypeStruct((Np, H), jnp.float32)),
    )
    u2, s2 = tc2(agg1, s1, dinv, b1.reshape(1, H), W2)

    agg2 = agg(u2, src, dst, zerosH)

    tc3 = pl.pallas_call(
        functools.partial(_tc3_body, N),
        out_shape=jax.ShapeDtypeStruct((1, O), jnp.float32),
    )
    out = tc3(agg2, s2, dinv, b2.reshape(1, H), Wfc, bfc.reshape(1, O))
    return out.reshape(O)


def _probe_single_agg(x, edge_index, W1, b1, W2, b2, Wfc, bfc):
    N, D = x.shape
    H = W1.shape[1]
    E = edge_index.shape[1]
    Np = ((N + 1 + 127) // 128) * 128
    blk = _NW * _C * _NB
    Ep = ((E + blk - 1) // blk) * blk
    rpt = Np // _NS
    npad = Ep - E
    pad_src = jnp.full((npad,), N, dtype=jnp.int32)
    pad_dst = N + (jnp.arange(npad, dtype=jnp.int32) % (Np - N))
    src = jnp.concatenate([edge_index[0], pad_src])
    dst = jnp.concatenate([edge_index[1], pad_dst])
    x_pad = jnp.zeros((Np, D), dtype=jnp.float32).at[:N].set(x)
    zerosH = jnp.zeros((rpt, H), dtype=jnp.float32)
    agg = _make_agg_kernel(Ep, Np, H)
    return agg(x_pad, src, dst, zerosH)


# speedup vs baseline: 1.9779x; 1.9779x over previous
import functools

import jax
import jax.numpy as jnp
from jax import lax
from jax.experimental import pallas as pl
from jax.experimental.pallas import tpu as pltpu
from jax.experimental.pallas import tpu_sc as plsc

_NC = 2
_NS = 16
_NW = _NC * _NS
_C = 88
_NB = 4
_NI = 8


def _sc_mesh():
    return plsc.VectorSubcoreMesh(core_axis_name="c", subcore_axis_name="s")


def _make_deg_kernel(Ep, Np):
    ept = Ep // _NW
    nv = ept // 16

    @functools.partial(
        pl.kernel,
        mesh=_sc_mesh(),
        compiler_params=pltpu.CompilerParams(needs_layout_passes=False),
        out_type=jax.ShapeDtypeStruct((_NW * Np,), jnp.float32),
        scratch_types=[
            pltpu.VMEM((ept,), jnp.int32),
            pltpu.VMEM((Np,), jnp.float32),
            pltpu.SemaphoreType.DMA,
        ],
    )
    def deg_kernel(dst_hbm, out_hbm, didx, hist, sem):
        cid = lax.axis_index("c")
        sid = lax.axis_index("s")
        wid = cid * _NS + sid
        pltpu.async_copy(dst_hbm.at[pl.ds(wid * ept, ept)], didx, sem).wait()
        zeros = jnp.zeros((16,), jnp.float32)

        def zbody(i, _):
            hist[pl.ds(i * 16, 16)] = zeros
            return 0

        lax.fori_loop(0, Np // 16, zbody, 0)

        def body(j, _):
            v = didx[pl.ds(j * 16, 16)]
            cnt, m = plsc.scan_count(v)
            plsc.addupdate_scatter(hist, [v], cnt.astype(jnp.float32), mask=m)
            return 0

        lax.fori_loop(0, nv, body, 0)
        pltpu.sync_copy(hist, out_hbm.at[pl.ds(wid * Np, Np)])

    return deg_kernel


def _make_agg_kernel(Ep, Np, H):
    ept = Ep // _NW
    nch = ept // _C
    rpt = Np // _NS

    @functools.partial(
        pl.kernel,
        mesh=_sc_mesh(),
        out_type=jax.ShapeDtypeStruct((_NC, Np, H), jnp.float32),
        scratch_types=[
            pltpu.VMEM((_NI, _C), jnp.int32),
            pltpu.VMEM((_NI, _C), jnp.int32),
            pltpu.VMEM((_NB, _C, H), jnp.float32),
            pltpu.VMEM_SHARED((Np, H), jnp.float32),
            [pltpu.SemaphoreType.DMA] * _NI,
            [pltpu.SemaphoreType.DMA] * _NI,
            [pltpu.SemaphoreType.DMA] * _NB,
        ],
    )
    def agg_kernel(u_hbm, src_hbm, dst_hbm, zeros_hbm, out_hbm,
                   sidx, didx, rows, acc, isems, dsems, gsems):
        cid = lax.axis_index("c")
        sid = lax.axis_index("s")
        wid = cid * _NS + sid
        row0 = sid * rpt

        ebase = wid * ept

        def load_idx(ci, slot):
            pltpu.async_copy(src_hbm.at[pl.ds(ebase + ci * _C, _C)],
                             sidx.at[slot], isems[slot])
            pltpu.async_copy(dst_hbm.at[pl.ds(ebase + ci * _C, _C)],
                             didx.at[slot], dsems[slot])

        def wait_sidx(ci, slot):
            pltpu.make_async_copy(src_hbm.at[pl.ds(ebase + ci * _C, _C)],
                                  sidx.at[slot], isems[slot]).wait()

        def wait_didx(ci, slot):
            pltpu.make_async_copy(dst_hbm.at[pl.ds(ebase + ci * _C, _C)],
                                  didx.at[slot], dsems[slot]).wait()

        def fire_gather(slot, gb):
            pltpu.async_copy(u_hbm.at[sidx.at[slot]], rows.at[gb], gsems[gb])

        def wait_gather(slot, gb):
            pltpu.make_async_copy(u_hbm.at[sidx.at[slot]], rows.at[gb],
                                  gsems[gb]).wait()

        for k in range(_NI - 1):
            load_idx(k, k)
        pltpu.sync_copy(zeros_hbm, acc.at[pl.ds(row0, rpt)])
        plsc.subcore_barrier()
        for k in range(_NB - 1):
            wait_sidx(k, k)
            fire_gather(k, k % _NB)

        def grp(g, _):
            for k in range(_NI):
                ci = g * _NI + k

                @pl.when(ci + _NI - 1 < nch)
                def _():
                    load_idx(ci + _NI - 1, (k + _NI - 1) % _NI)

                @pl.when(ci + _NB - 1 < nch)
                def _():
                    wait_sidx(ci + _NB - 1, (k + _NB - 1) % _NI)
                    fire_gather((k + _NB - 1) % _NI, (k + _NB - 1) % _NB)
                wait_gather(k % _NI, k % _NB)
                wait_didx(ci, k % _NI)
                pltpu.sync_copy(rows.at[k % _NB], acc.at[didx.at[k % _NI]],
                                add=True)
            return 0

        lax.fori_loop(0, nch // _NI, grp, 0)
        plsc.subcore_barrier()
        pltpu.sync_copy(acc.at[pl.ds(row0, rpt)], out_hbm.at[cid, pl.ds(row0, rpt)])

    return agg_kernel


def _tc1_body(x_ref, w_ref, hists_ref, ones_ref, u_ref, s_ref, dinv_ref):
    deg = lax.dot_general(hists_ref[...], ones_ref[...],
                          (((0,), (0,)), ((), ())),
                          preferred_element_type=jnp.float32) + 1.0
    dinv = lax.rsqrt(deg)
    h = jnp.dot(x_ref[...], w_ref[...], preferred_element_type=jnp.float32)
    u = h * dinv
    u_ref[...] = u
    s_ref[...] = u * dinv
    dinv_ref[...] = dinv


def _tc2_body(aggp_ref, s_ref, dinv_ref, b_ref, w_ref, u_ref, s2_ref):
    dinv = dinv_ref[...]
    z = dinv * (aggp_ref[0] + aggp_ref[1]) + s_ref[...] + b_ref[...]
    r = jnp.maximum(z, 0.0)
    h = jnp.dot(r, w_ref[...], preferred_element_type=jnp.float32)
    u = h * dinv
    u_ref[...] = u
    s2_ref[...] = u * dinv


def _tc3_body(n_real, aggp_ref, s_ref, dinv_ref, b_ref, wfc_ref, bfc_ref, out_ref):
    z = dinv_ref[...] * (aggp_ref[0] + aggp_ref[1]) + s_ref[...] + b_ref[...]
    r = jnp.maximum(z, 0.0)
    rows = lax.broadcasted_iota(jnp.int32, r.shape, 0)
    r = jnp.where(rows < n_real, r, 0.0)
    g = jnp.sum(r, axis=0, keepdims=True) * (1.0 / n_real)
    out_ref[...] = jnp.dot(g, wfc_ref[...],
                           preferred_element_type=jnp.float32) + bfc_ref[...]


def kernel(x, edge_index, W1, b1, W2, b2, Wfc, bfc):
    N, D = x.shape
    H = W1.shape[1]
    O = Wfc.shape[1]
    E = edge_index.shape[1]

    Np = ((N + 1 + 127) // 128) * 128
    blk = _NW * _C * _NB
    Ep = ((E + blk - 1) // blk) * blk
    rpt = Np // _NS
    nch = Ep // (_NW * _C)

    npad = Ep - E
    pad_src = N + (jnp.arange(npad, dtype=jnp.int32) % (Np - N))
    pad_dst = N + (jnp.arange(npad, dtype=jnp.int32) % (Np - N))
    src = jnp.concatenate([edge_index[0], pad_src])
    dst = jnp.concatenate([edge_index[1], pad_dst])
    x_pad = jnp.zeros((Np, D), dtype=jnp.float32).at[:N].set(x)
    ones32 = jnp.ones((_NW, 1), dtype=jnp.float32)
    zerosH = jnp.zeros((rpt, H), dtype=jnp.float32)

    hists = _make_deg_kernel(Ep, Np)(dst).reshape(_NW, Np)

    tc1 = pl.pallas_call(
        _tc1_body,
        out_shape=(jax.ShapeDtypeStruct((Np, H), jnp.float32),
                   jax.ShapeDtypeStruct((Np, H), jnp.float32),
                   jax.ShapeDtypeStruct((Np, 1), jnp.float32)),
    )
    u1, s1, dinv = tc1(x_pad, W1, hists, ones32)

    agg = _make_agg_kernel(Ep, Np, H)
    agg1 = agg(u1, src, dst, zerosH)

    tc2 = pl.pallas_call(
        _tc2_body,
        out_shape=(jax.ShapeDtypeStruct((Np, H), jnp.float32),
                   jax.ShapeDtypeStruct((Np, H), jnp.float32)),
    )
    u2, s2 = tc2(agg1, s1, dinv, b1.reshape(1, H), W2)

    agg2 = agg(u2, src, dst, zerosH)

    tc3 = pl.pallas_call(
        functools.partial(_tc3_body, N),
        out_shape=jax.ShapeDtypeStruct((1, O), jnp.float32),
    )
    out = tc3(agg2, s2, dinv, b2.reshape(1, H), Wfc, bfc.reshape(1, O))
    return out.reshape(O)


def _probe_single_agg(x, edge_index, W1, b1, W2, b2, Wfc, bfc):
    N, D = x.shape
    H = W1.shape[1]
    E = edge_index.shape[1]
    Np = ((N + 1 + 127) // 128) * 128
    blk = _NW * _C * _NB
    Ep = ((E + blk - 1) // blk) * blk
    rpt = Np // _NS
    npad = Ep - E
    pad_src = N + (jnp.arange(npad, dtype=jnp.int32) % (Np - N))
    pad_dst = N + (jnp.arange(npad, dtype=jnp.int32) % (Np - N))
    src = jnp.concatenate([edge_index[0], pad_src])
    dst = jnp.concatenate([edge_index[1], pad_dst])
    x_pad = jnp.zeros((Np, D), dtype=jnp.float32).at[:N].set(x)
    zerosH = jnp.zeros((rpt, H), dtype=jnp.float32)
    agg = _make_agg_kernel(Ep, Np, H)
    return agg(x_pad, src, dst, zerosH)
